# Initial kernel scaffold; baseline (speedup 1.0000x reference)
import jax
import jax.numpy as jnp
from jax.experimental import pallas as pl


def kernel(node_attr, graph, params):
    raise NotImplementedError("write your pallas kernel here")


# speedup vs baseline: 2.5305x; 2.5305x over previous
import jax
import jax.numpy as jnp
from jax import lax
from jax.experimental import pallas as pl
from jax.experimental.pallas import tpu as pltpu
from jax.experimental.pallas import tpu_sc as plsc

N_NODES = 10000
N_EDGES = 320000
D = 128
_f32 = jnp.float32

_NC, _NS = 2, 16
_NW = _NC * _NS
_EPW = N_EDGES // _NW
_CH = 80
_NCH = _EPW // _CH

def _sc_mesh():
    return plsc.VectorSubcoreMesh(
        core_axis_name="c", subcore_axis_name="s", num_cores=_NC, num_subcores=_NS
    )


def _gather2_body(p_hbm, q_hbm, src_hbm, dst_hbm, gs_hbm, gd_hbm,
                  idx_s, idx_d, rows_s, rows_d, sem_s, sem_d):
    c = lax.axis_index("c")
    s = lax.axis_index("s")
    base = (s * _NC + c) * _EPW

    def chunk(j, carry):
        off = base + j * _CH
        pltpu.sync_copy(src_hbm.at[pl.ds(off, _CH)], idx_s)
        pltpu.sync_copy(dst_hbm.at[pl.ds(off, _CH)], idx_d)
        cps = pltpu.async_copy(p_hbm.at[idx_s], rows_s, sem_s)
        cpd = pltpu.async_copy(q_hbm.at[idx_d], rows_d, sem_d)
        cps.wait()
        cpd.wait()
        pltpu.sync_copy(rows_s, gs_hbm.at[pl.ds(off, _CH)])
        pltpu.sync_copy(rows_d, gd_hbm.at[pl.ds(off, _CH)])
        return carry

    lax.fori_loop(0, _NCH, chunk, 0)


def _gather2(p, q, src, dst):
    f = pl.kernel(
        _gather2_body,
        out_type=(jax.ShapeDtypeStruct((N_EDGES, D), _f32),
                  jax.ShapeDtypeStruct((N_EDGES, D), _f32)),
        mesh=_sc_mesh(),
        scratch_types=(
            pltpu.VMEM((_CH,), jnp.int32),
            pltpu.VMEM((_CH,), jnp.int32),
            pltpu.VMEM((_CH, D), _f32),
            pltpu.VMEM((_CH, D), _f32),
            pltpu.SemaphoreType.DMA,
            pltpu.SemaphoreType.DMA,
        ),
    )
    return f(p, q, src, dst)


def _scatter_body(edges_hbm, dst_hbm, zeros_hbm, out_hbm, idx_v, rows_v, acc):
    c = lax.axis_index("c")
    s = lax.axis_index("s")
    base = (c * _NS + s) * _EPW

    @pl.when(s == 0)
    def _():
        pltpu.sync_copy(zeros_hbm, acc)

    plsc.subcore_barrier()

    def chunk(j, carry):
        off = base + j * _CH
        pltpu.sync_copy(dst_hbm.at[pl.ds(off, _CH)], idx_v)
        pltpu.sync_copy(edges_hbm.at[pl.ds(off, _CH)], rows_v)
        pltpu.sync_copy(rows_v, acc.at[idx_v], add=True)
        return carry

    lax.fori_loop(0, _NCH, chunk, 0)

    plsc.subcore_barrier()

    @pl.when(s == 0)
    def _():
        pltpu.sync_copy(acc, out_hbm.at[c])


def _scatter_add(edges, dst, zeros):
    f = pl.kernel(
        _scatter_body,
        out_type=jax.ShapeDtypeStruct((_NC, N_NODES, D), _f32),
        mesh=_sc_mesh(),
        scratch_types=(
            pltpu.VMEM((_CH,), jnp.int32),
            pltpu.VMEM((_CH, D), _f32),
            pltpu.VMEM_SHARED((N_NODES, D), _f32),
        ),
    )
    return f(edges, dst, zeros)


_NBLK = 1000
_NGRID = N_NODES // _NBLK
_EBLK = 2560
_EGRID = N_EDGES // _EBLK


def _row_spec(blk):
    return pl.BlockSpec((blk, D), lambda i: (i, 0))


def _w_spec():
    return pl.BlockSpec((D, D), lambda i: (0, 0))


def _b_spec():
    return pl.BlockSpec((1, D), lambda i: (0, 0))


def _dot(a, b):
    return jnp.dot(a, b, preferred_element_type=_f32)


def _enc_body(x_ref, w1_ref, b1_ref, w2_ref, b2_ref, a_ref, b_ref,
              nodes_ref, p_ref, q_ref):
    h = jax.nn.gelu(_dot(x_ref[...], w1_ref[...]) + b1_ref[...])
    n = _dot(h, w2_ref[...]) + b2_ref[...]
    nodes_ref[...] = n
    p_ref[...] = _dot(n, a_ref[...])
    q_ref[...] = _dot(n, b_ref[...])


def _node_enc(x, w1, b1, w2, b2, a, b):
    return pl.pallas_call(
        _enc_body,
        grid=(_NGRID,),
        in_specs=[_row_spec(_NBLK), _w_spec(), _b_spec(), _w_spec(), _b_spec(),
                  _w_spec(), _w_spec()],
        out_specs=[_row_spec(_NBLK)] * 3,
        out_shape=[jax.ShapeDtypeStruct((N_NODES, D), _f32)] * 3,
    )(x, w1, b1, w2, b2, a, b)


def _node_layer_body(n_ref, agg_ref, wn_ref, wa_ref, b1_ref, w2_ref, b2_ref,
                     a_ref, b_ref, out_ref, p_ref, q_ref):
    n = n_ref[...]
    agg = agg_ref[0] + agg_ref[1]
    h = jax.nn.gelu(_dot(n, wn_ref[...]) + _dot(agg, wa_ref[...]) + b1_ref[...])
    nn = _dot(h, w2_ref[...]) + b2_ref[...] + n
    out_ref[...] = nn
    p_ref[...] = _dot(nn, a_ref[...])
    q_ref[...] = _dot(nn, b_ref[...])


def _node_layer(nodes, agg2, wn, wa, b1, w2, b2, a, b):
    return pl.pallas_call(
        _node_layer_body,
        grid=(_NGRID,),
        in_specs=[_row_spec(_NBLK),
                  pl.BlockSpec((_NC, _NBLK, D), lambda i: (0, i, 0)),
                  _w_spec(), _w_spec(), _b_spec(), _w_spec(), _b_spec(),
                  _w_spec(), _w_spec()],
        out_specs=[_row_spec(_NBLK)] * 3,
        out_shape=[jax.ShapeDtypeStruct((N_NODES, D), _f32)] * 3,
    )(nodes, agg2, wn, wa, b1, w2, b2, a, b)


def _edge_enc_body(gs_ref, gd_ref, b1_ref, w2_ref, b2_ref, out_ref):
    h = jax.nn.gelu(gs_ref[...] + gd_ref[...] + b1_ref[...])
    out_ref[...] = _dot(h, w2_ref[...]) + b2_ref[...]


def _edge_enc(gs, gd, b1, w2, b2):
    return pl.pallas_call(
        _edge_enc_body,
        grid=(_EGRID,),
        in_specs=[_row_spec(_EBLK), _row_spec(_EBLK), _b_spec(), _w_spec(),
                  _b_spec()],
        out_specs=_row_spec(_EBLK),
        out_shape=jax.ShapeDtypeStruct((N_EDGES, D), _f32),
    )(gs, gd, b1, w2, b2)


def _edge_layer_body(e_ref, gs_ref, gd_ref, c_ref, b1_ref, w2_ref, b2_ref,
                     out_ref):
    e = e_ref[...]
    h = jax.nn.gelu(gs_ref[...] + gd_ref[...] + _dot(e, c_ref[...]) + b1_ref[...])
    out_ref[...] = _dot(h, w2_ref[...]) + b2_ref[...] + e


def _edge_layer(edges, gs, gd, c, b1, w2, b2):
    return pl.pallas_call(
        _edge_layer_body,
        grid=(_EGRID,),
        in_specs=[_row_spec(_EBLK), _row_spec(_EBLK), _row_spec(_EBLK),
                  _w_spec(), _b_spec(), _w_spec(), _b_spec()],
        out_specs=_row_spec(_EBLK),
        out_shape=jax.ShapeDtypeStruct((N_EDGES, D), _f32),
    )(edges, gs, gd, c, b1, w2, b2)


def kernel(node_attr, graph, params):
    src = graph[0].astype(jnp.int32)
    dst = graph[1].astype(jnp.int32)
    enc = params["node_enc"]
    ee = params["edge_enc"]

    nodes, p, q = _node_enc(
        node_attr, enc["W1"], enc["b1"].reshape(1, D), enc["W2"],
        enc["b2"].reshape(1, D), ee["W1"][:D], ee["W1"][D:])
    gs, gd = _gather2(p, q, src, dst)
    edges = _edge_enc(gs, gd, ee["b1"].reshape(1, D), ee["W2"],
                      ee["b2"].reshape(1, D))

    zeros = jnp.zeros((N_NODES, D), _f32)
    for lp in params["layers"]:
        np_ = lp["node_net"]
        ep_ = lp["edge_net"]
        agg2 = _scatter_add(edges, dst, zeros)
        nodes, p, q = _node_layer(
            nodes, agg2, np_["W1"][:D], np_["W1"][D:],
            np_["b1"].reshape(1, D), np_["W2"], np_["b2"].reshape(1, D),
            ep_["W1"][:D], ep_["W1"][D:2 * D])
        gs, gd = _gather2(p, q, src, dst)
        edges = _edge_layer(edges, gs, gd, ep_["W1"][2 * D:],
                            ep_["b1"].reshape(1, D), ep_["W2"],
                            ep_["b2"].reshape(1, D))
    return (nodes, edges)


# speedup vs baseline: 3.7188x; 1.4696x over previous
import jax
import jax.numpy as jnp
from jax import lax
from jax.experimental import pallas as pl
from jax.experimental.pallas import tpu as pltpu
from jax.experimental.pallas import tpu_sc as plsc

N_NODES = 10000
N_EDGES = 320000
D = 128
_f32 = jnp.float32

_NC, _NS = 2, 16
_NW = _NC * _NS
_EPW = N_EDGES // _NW
_CH = 80
_NCH = _EPW // _CH

def _sc_mesh():
    return plsc.VectorSubcoreMesh(
        core_axis_name="c", subcore_axis_name="s", num_cores=_NC, num_subcores=_NS
    )


_K = 5
_CHUNK = _CH * _K
_NCHK = _EPW // _CHUNK
_IPT = _EPW // _CH


def _gather_pass(tbl_hbm, out_hbm, idx2d, bufA, bufB, semA, semB,
                 semwA, semwB, base):
    def g_start(ch, buf, sem):
        for k in range(_K):
            pltpu.async_copy(tbl_hbm.at[idx2d.at[ch * _K + k]],
                             buf.at[pl.ds(k * _CH, _CH)], sem)

    def g_wait(buf, sem):
        pltpu.make_async_copy(out_hbm.at[pl.ds(0, _CHUNK)], buf, sem).wait()

    def wb_start(ch, buf, semw):
        pltpu.async_copy(buf, out_hbm.at[pl.ds(base + ch * _CHUNK, _CHUNK)],
                         semw)

    def wb_wait(buf, semw):
        pltpu.make_async_copy(buf, out_hbm.at[pl.ds(0, _CHUNK)], semw).wait()

    g_start(0, bufA, semA)
    g_start(1, bufB, semB)

    def body(t, carry):
        c0 = 2 * t
        g_wait(bufA, semA)
        wb_start(c0, bufA, semwA)
        g_wait(bufB, semB)
        wb_start(c0 + 1, bufB, semwB)
        wb_wait(bufA, semwA)
        g_start(c0 + 2, bufA, semA)

        @pl.when(c0 + 3 < _NCHK)
        def _():
            wb_wait(bufB, semwB)
            g_start(c0 + 3, bufB, semB)

        return carry

    lax.fori_loop(0, _NCHK // 2, body, 0)
    g_wait(bufA, semA)
    wb_start(_NCHK - 1, bufA, semwA)
    wb_wait(bufB, semwB)
    wb_wait(bufA, semwA)


def _gather2_body(p_hbm, q_hbm, src2d_hbm, dst2d_hbm, gs_hbm, gd_hbm,
                  idx, bufA, bufB, semA, semB, semwA, semwB):
    c = lax.axis_index("c")
    s = lax.axis_index("s")
    wid = s * _NC + c
    base = wid * _EPW
    pltpu.sync_copy(src2d_hbm.at[wid], idx)
    _gather_pass(p_hbm, gs_hbm, idx, bufA, bufB, semA, semB, semwA, semwB,
                 base)
    pltpu.sync_copy(dst2d_hbm.at[wid], idx)
    _gather_pass(q_hbm, gd_hbm, idx, bufA, bufB, semA, semB, semwA, semwB,
                 base)


def _gather2(p, q, src2d, dst2d):
    f = pl.kernel(
        _gather2_body,
        out_type=(jax.ShapeDtypeStruct((N_EDGES, D), _f32),
                  jax.ShapeDtypeStruct((N_EDGES, D), _f32)),
        mesh=_sc_mesh(),
        scratch_types=(
            pltpu.VMEM((_IPT, _CH), jnp.int32),
            pltpu.VMEM((_CHUNK, D), _f32),
            pltpu.VMEM((_CHUNK, D), _f32),
            pltpu.SemaphoreType.DMA,
            pltpu.SemaphoreType.DMA,
            pltpu.SemaphoreType.DMA,
            pltpu.SemaphoreType.DMA,
        ),
    )
    return f(p, q, src2d, dst2d)


def _scatter_body(edges_hbm, dst2d_hbm, zeros_hbm, out_hbm,
                  idx_d, bufA, bufB, acc, semA, semB):
    c = lax.axis_index("c")
    s = lax.axis_index("s")
    wid = c * _NS + s
    base = wid * _EPW

    pltpu.sync_copy(dst2d_hbm.at[wid], idx_d)

    @pl.when(s == 0)
    def _():
        pltpu.sync_copy(zeros_hbm, acc)

    plsc.subcore_barrier()

    def r_start(ch, buf, sem):
        pltpu.async_copy(edges_hbm.at[pl.ds(base + ch * _CH, _CH)], buf, sem)

    def r_wait(buf, sem):
        pltpu.make_async_copy(edges_hbm.at[pl.ds(0, _CH)], buf, sem).wait()

    def scat(ch, buf):
        pltpu.sync_copy(buf, acc.at[idx_d.at[ch]], add=True)

    r_start(0, bufA, semA)
    r_start(1, bufB, semB)

    def body(t, carry):
        c0 = 2 * t
        r_wait(bufA, semA)
        scat(c0, bufA)
        r_start(c0 + 2, bufA, semA)
        r_wait(bufB, semB)
        scat(c0 + 1, bufB)

        @pl.when(c0 + 3 < _IPT)
        def _():
            r_start(c0 + 3, bufB, semB)

        return carry

    lax.fori_loop(0, _IPT // 2, body, 0)
    r_wait(bufA, semA)
    scat(_IPT - 1, bufA)

    plsc.subcore_barrier()

    @pl.when(s == 0)
    def _():
        pltpu.sync_copy(acc, out_hbm.at[c])


def _scatter_add(edges, dst2d, zeros):
    f = pl.kernel(
        _scatter_body,
        out_type=jax.ShapeDtypeStruct((_NC, N_NODES, D), _f32),
        mesh=_sc_mesh(),
        scratch_types=(
            pltpu.VMEM((_IPT, _CH), jnp.int32),
            pltpu.VMEM((_CH, D), _f32),
            pltpu.VMEM((_CH, D), _f32),
            pltpu.VMEM_SHARED((N_NODES, D), _f32),
            pltpu.SemaphoreType.DMA,
            pltpu.SemaphoreType.DMA,
        ),
    )
    return f(edges, dst2d, zeros)


_NBLK = 1000
_NGRID = N_NODES // _NBLK
_EBLK = 2560
_EGRID = N_EDGES // _EBLK


def _row_spec(blk):
    return pl.BlockSpec((blk, D), lambda i: (i, 0))


def _w_spec():
    return pl.BlockSpec((D, D), lambda i: (0, 0))


def _b_spec():
    return pl.BlockSpec((1, D), lambda i: (0, 0))


def _dot(a, b):
    return jnp.dot(a, b, preferred_element_type=_f32)


def _enc_body(x_ref, w1_ref, b1_ref, w2_ref, b2_ref, a_ref, b_ref,
              nodes_ref, p_ref, q_ref):
    h = jax.nn.gelu(_dot(x_ref[...], w1_ref[...]) + b1_ref[...])
    n = _dot(h, w2_ref[...]) + b2_ref[...]
    nodes_ref[...] = n
    p_ref[...] = _dot(n, a_ref[...])
    q_ref[...] = _dot(n, b_ref[...])


def _node_enc(x, w1, b1, w2, b2, a, b):
    return pl.pallas_call(
        _enc_body,
        grid=(_NGRID,),
        in_specs=[_row_spec(_NBLK), _w_spec(), _b_spec(), _w_spec(), _b_spec(),
                  _w_spec(), _w_spec()],
        out_specs=[_row_spec(_NBLK)] * 3,
        out_shape=[jax.ShapeDtypeStruct((N_NODES, D), _f32)] * 3,
    )(x, w1, b1, w2, b2, a, b)


def _node_layer_body(n_ref, agg_ref, wn_ref, wa_ref, b1_ref, w2_ref, b2_ref,
                     a_ref, b_ref, out_ref, p_ref, q_ref):
    n = n_ref[...]
    agg = agg_ref[0] + agg_ref[1]
    h = jax.nn.gelu(_dot(n, wn_ref[...]) + _dot(agg, wa_ref[...]) + b1_ref[...])
    nn = _dot(h, w2_ref[...]) + b2_ref[...] + n
    out_ref[...] = nn
    p_ref[...] = _dot(nn, a_ref[...])
    q_ref[...] = _dot(nn, b_ref[...])


def _node_layer(nodes, agg2, wn, wa, b1, w2, b2, a, b):
    return pl.pallas_call(
        _node_layer_body,
        grid=(_NGRID,),
        in_specs=[_row_spec(_NBLK),
                  pl.BlockSpec((_NC, _NBLK, D), lambda i: (0, i, 0)),
                  _w_spec(), _w_spec(), _b_spec(), _w_spec(), _b_spec(),
                  _w_spec(), _w_spec()],
        out_specs=[_row_spec(_NBLK)] * 3,
        out_shape=[jax.ShapeDtypeStruct((N_NODES, D), _f32)] * 3,
    )(nodes, agg2, wn, wa, b1, w2, b2, a, b)


def _edge_enc_body(gs_ref, gd_ref, b1_ref, w2_ref, b2_ref, out_ref):
    h = jax.nn.gelu(gs_ref[...] + gd_ref[...] + b1_ref[...])
    out_ref[...] = _dot(h, w2_ref[...]) + b2_ref[...]


def _edge_enc(gs, gd, b1, w2, b2):
    return pl.pallas_call(
        _edge_enc_body,
        grid=(_EGRID,),
        in_specs=[_row_spec(_EBLK), _row_spec(_EBLK), _b_spec(), _w_spec(),
                  _b_spec()],
        out_specs=_row_spec(_EBLK),
        out_shape=jax.ShapeDtypeStruct((N_EDGES, D), _f32),
    )(gs, gd, b1, w2, b2)


def _edge_layer_body(e_ref, gs_ref, gd_ref, c_ref, b1_ref, w2_ref, b2_ref,
                     out_ref):
    e = e_ref[...]
    h = jax.nn.gelu(gs_ref[...] + gd_ref[...] + _dot(e, c_ref[...]) + b1_ref[...])
    out_ref[...] = _dot(h, w2_ref[...]) + b2_ref[...] + e


def _edge_layer(edges, gs, gd, c, b1, w2, b2):
    return pl.pallas_call(
        _edge_layer_body,
        grid=(_EGRID,),
        in_specs=[_row_spec(_EBLK), _row_spec(_EBLK), _row_spec(_EBLK),
                  _w_spec(), _b_spec(), _w_spec(), _b_spec()],
        out_specs=_row_spec(_EBLK),
        out_shape=jax.ShapeDtypeStruct((N_EDGES, D), _f32),
    )(edges, gs, gd, c, b1, w2, b2)


def kernel(node_attr, graph, params):
    src2d = graph[0].astype(jnp.int32).reshape(_NW, _IPT, _CH)
    dst2d = graph[1].astype(jnp.int32).reshape(_NW, _IPT, _CH)
    enc = params["node_enc"]
    ee = params["edge_enc"]

    nodes, p, q = _node_enc(
        node_attr, enc["W1"], enc["b1"].reshape(1, D), enc["W2"],
        enc["b2"].reshape(1, D), ee["W1"][:D], ee["W1"][D:])
    gs, gd = _gather2(p, q, src2d, dst2d)
    edges = _edge_enc(gs, gd, ee["b1"].reshape(1, D), ee["W2"],
                      ee["b2"].reshape(1, D))

    zeros = jnp.zeros((N_NODES, D), _f32)
    for lp in params["layers"]:
        np_ = lp["node_net"]
        ep_ = lp["edge_net"]
        agg2 = _scatter_add(edges, dst2d, zeros)
        nodes, p, q = _node_layer(
            nodes, agg2, np_["W1"][:D], np_["W1"][D:],
            np_["b1"].reshape(1, D), np_["W2"], np_["b2"].reshape(1, D),
            ep_["W1"][:D], ep_["W1"][D:2 * D])
        gs, gd = _gather2(p, q, src2d, dst2d)
        edges = _edge_layer(edges, gs, gd, ep_["W1"][2 * D:],
                            ep_["b1"].reshape(1, D), ep_["W2"],
                            ep_["b2"].reshape(1, D))
    return (nodes, edges)
